# baseline (speedup 1.0000x reference)
import jax
import jax.numpy as jnp
from jax import lax
from jax.experimental import pallas as pl
from jax.experimental.pallas import tpu as pltpu

EMBED_DIM = 2048
NUM_EXPERTS = 16
N_TOKENS = 16384

NBUF = 4
CH = 1024
NST = N_TOKENS // CH


def _gate_body(x_hbm, w_ref, b_ref, gates_ref, idx_ref, xbuf, sems):
    w = w_ref[...]
    b2 = b_ref[...]
    HC = CH // 2

    def start(i, slot):
        pltpu.make_async_copy(
            x_hbm.at[pl.ds(i * CH, HC)], xbuf.at[slot, pl.ds(0, HC)],
            sems.at[slot, 0]).start()
        pltpu.make_async_copy(
            x_hbm.at[pl.ds(i * CH + HC, HC)], xbuf.at[slot, pl.ds(HC, HC)],
            sems.at[slot, 1]).start()

    def wait(i, slot):
        pltpu.make_async_copy(
            x_hbm.at[pl.ds(i * CH, HC)], xbuf.at[slot, pl.ds(0, HC)],
            sems.at[slot, 0]).wait()
        pltpu.make_async_copy(
            x_hbm.at[pl.ds(i * CH + HC, HC)], xbuf.at[slot, pl.ds(HC, HC)],
            sems.at[slot, 1]).wait()

    for i in range(NBUF):
        start(i, i)

    def step(i, _):
        slot = lax.rem(i, NBUF)
        wait(i, slot)
        logits = jax.lax.dot_general(
            w, xbuf[slot], (((1,), (1,)), ((), ())),
            preferred_element_type=jnp.float32) + b2

        rows = jax.lax.broadcasted_iota(jnp.int32, logits.shape, 0)
        m1 = jnp.max(logits, axis=0, keepdims=True)
        i1 = jnp.min(jnp.where(logits == m1, rows, NUM_EXPERTS),
                     axis=0, keepdims=True)
        masked = jnp.where(rows == i1, -jnp.inf, logits)
        m2 = jnp.max(masked, axis=0, keepdims=True)
        i2 = jnp.min(jnp.where(masked == m2, rows, NUM_EXPERTS),
                     axis=0, keepdims=True)
        e2 = jnp.exp(m2 - m1)
        den = 1.0 + e2
        g = jnp.concatenate([1.0 / den, e2 / den], axis=0)
        ix = jnp.concatenate([i1, i2], axis=0)
        gates_ref[:, pl.ds(i * CH, CH)] = g
        idx_ref[:, pl.ds(i * CH, CH)] = ix

        @pl.when(i + NBUF < NST)
        def _():
            start(i + NBUF, slot)

        return 0

    lax.fori_loop(0, NST, step, 0)


def kernel(x, W, b):
    gates_t, idx_t = pl.pallas_call(
        _gate_body,
        in_specs=[
            pl.BlockSpec(memory_space=pl.ANY),
            pl.BlockSpec((NUM_EXPERTS, EMBED_DIM), lambda: (0, 0)),
            pl.BlockSpec((NUM_EXPERTS, 1), lambda: (0, 0)),
        ],
        out_specs=[
            pl.BlockSpec((2, N_TOKENS), lambda: (0, 0)),
            pl.BlockSpec((2, N_TOKENS), lambda: (0, 0)),
        ],
        out_shape=[
            jax.ShapeDtypeStruct((2, N_TOKENS), jnp.float32),
            jax.ShapeDtypeStruct((2, N_TOKENS), jnp.int32),
        ],
        scratch_shapes=[
            pltpu.VMEM((NBUF, CH, EMBED_DIM), jnp.float32),
            pltpu.SemaphoreType.DMA((NBUF, 2)),
        ],
    )(x, W, b.reshape(NUM_EXPERTS, 1))
    return (gates_t.T, idx_t.T)


# speedup vs baseline: 1.0375x; 1.0375x over previous
import jax
import jax.numpy as jnp
from jax import lax
from jax.experimental import pallas as pl
from jax.experimental.pallas import tpu as pltpu

EMBED_DIM = 2048
NUM_EXPERTS = 16
N_TOKENS = 16384

NBUF = 8
CH = 256
NST = N_TOKENS // CH


def _gate_body(x_hbm, w_ref, b_ref, gates_ref, idx_ref, xbuf, sems):
    w = w_ref[...]
    b2 = b_ref[...]
    HC = CH // 2

    def start(i, slot):
        pltpu.make_async_copy(
            x_hbm.at[pl.ds(i * CH, HC)], xbuf.at[slot, pl.ds(0, HC)],
            sems.at[slot, 0]).start()
        pltpu.make_async_copy(
            x_hbm.at[pl.ds(i * CH + HC, HC)], xbuf.at[slot, pl.ds(HC, HC)],
            sems.at[slot, 1]).start()

    def wait(i, slot):
        pltpu.make_async_copy(
            x_hbm.at[pl.ds(i * CH, HC)], xbuf.at[slot, pl.ds(0, HC)],
            sems.at[slot, 0]).wait()
        pltpu.make_async_copy(
            x_hbm.at[pl.ds(i * CH + HC, HC)], xbuf.at[slot, pl.ds(HC, HC)],
            sems.at[slot, 1]).wait()

    for i in range(NBUF):
        start(i, i)

    def step(i, _):
        slot = lax.rem(i, NBUF)
        wait(i, slot)
        logits = jax.lax.dot_general(
            w, xbuf[slot], (((1,), (1,)), ((), ())),
            preferred_element_type=jnp.float32) + b2

        rows = jax.lax.broadcasted_iota(jnp.int32, logits.shape, 0)
        m1 = jnp.max(logits, axis=0, keepdims=True)
        i1 = jnp.min(jnp.where(logits == m1, rows, NUM_EXPERTS),
                     axis=0, keepdims=True)
        masked = jnp.where(rows == i1, -jnp.inf, logits)
        m2 = jnp.max(masked, axis=0, keepdims=True)
        i2 = jnp.min(jnp.where(masked == m2, rows, NUM_EXPERTS),
                     axis=0, keepdims=True)
        e2 = jnp.exp(m2 - m1)
        den = 1.0 + e2
        g = jnp.concatenate([1.0 / den, e2 / den], axis=0)
        ix = jnp.concatenate([i1, i2], axis=0)
        gates_ref[:, pl.ds(i * CH, CH)] = g
        idx_ref[:, pl.ds(i * CH, CH)] = ix

        @pl.when(i + NBUF < NST)
        def _():
            start(i + NBUF, slot)

        return 0

    lax.fori_loop(0, NST, step, 0)


def kernel(x, W, b):
    gates_t, idx_t = pl.pallas_call(
        _gate_body,
        in_specs=[
            pl.BlockSpec(memory_space=pl.ANY),
            pl.BlockSpec((NUM_EXPERTS, EMBED_DIM), lambda: (0, 0)),
            pl.BlockSpec((NUM_EXPERTS, 1), lambda: (0, 0)),
        ],
        out_specs=[
            pl.BlockSpec((2, N_TOKENS), lambda: (0, 0)),
            pl.BlockSpec((2, N_TOKENS), lambda: (0, 0)),
        ],
        out_shape=[
            jax.ShapeDtypeStruct((2, N_TOKENS), jnp.float32),
            jax.ShapeDtypeStruct((2, N_TOKENS), jnp.int32),
        ],
        scratch_shapes=[
            pltpu.VMEM((NBUF, CH, EMBED_DIM), jnp.float32),
            pltpu.SemaphoreType.DMA((NBUF, 2)),
        ],
    )(x, W, b.reshape(NUM_EXPERTS, 1))
    return (gates_t.T, idx_t.T)


# speedup vs baseline: 1.0426x; 1.0049x over previous
import jax
import jax.numpy as jnp
from jax import lax
from jax.experimental import pallas as pl
from jax.experimental.pallas import tpu as pltpu

EMBED_DIM = 2048
NUM_EXPERTS = 16
N_TOKENS = 16384

NBUF = 6
CH = 512
NST = N_TOKENS // CH


def _gate_body(x_hbm, w_ref, b_ref, gates_ref, idx_ref, xbuf, sems):
    w = w_ref[...]
    b2 = b_ref[...]
    HC = CH // 2

    def start(i, slot):
        pltpu.make_async_copy(
            x_hbm.at[pl.ds(i * CH, HC)], xbuf.at[slot, pl.ds(0, HC)],
            sems.at[slot, 0]).start()
        pltpu.make_async_copy(
            x_hbm.at[pl.ds(i * CH + HC, HC)], xbuf.at[slot, pl.ds(HC, HC)],
            sems.at[slot, 1]).start()

    def wait(i, slot):
        pltpu.make_async_copy(
            x_hbm.at[pl.ds(i * CH, HC)], xbuf.at[slot, pl.ds(0, HC)],
            sems.at[slot, 0]).wait()
        pltpu.make_async_copy(
            x_hbm.at[pl.ds(i * CH + HC, HC)], xbuf.at[slot, pl.ds(HC, HC)],
            sems.at[slot, 1]).wait()

    for i in range(NBUF):
        start(i, i)

    def step(i, _):
        slot = lax.rem(i, NBUF)
        wait(i, slot)
        logits = jax.lax.dot_general(
            w, xbuf[slot], (((1,), (1,)), ((), ())),
            preferred_element_type=jnp.float32) + b2

        rows = jax.lax.broadcasted_iota(jnp.int32, logits.shape, 0)
        m1 = jnp.max(logits, axis=0, keepdims=True)
        i1 = jnp.min(jnp.where(logits == m1, rows, NUM_EXPERTS),
                     axis=0, keepdims=True)
        masked = jnp.where(rows == i1, -jnp.inf, logits)
        m2 = jnp.max(masked, axis=0, keepdims=True)
        i2 = jnp.min(jnp.where(masked == m2, rows, NUM_EXPERTS),
                     axis=0, keepdims=True)
        e2 = jnp.exp(m2 - m1)
        den = 1.0 + e2
        g = jnp.concatenate([1.0 / den, e2 / den], axis=0)
        ix = jnp.concatenate([i1, i2], axis=0)
        gates_ref[:, pl.ds(i * CH, CH)] = g
        idx_ref[:, pl.ds(i * CH, CH)] = ix

        @pl.when(i + NBUF < NST)
        def _():
            start(i + NBUF, slot)

        return 0

    lax.fori_loop(0, NST, step, 0)


def kernel(x, W, b):
    gates_t, idx_t = pl.pallas_call(
        _gate_body,
        in_specs=[
            pl.BlockSpec(memory_space=pl.ANY),
            pl.BlockSpec((NUM_EXPERTS, EMBED_DIM), lambda: (0, 0)),
            pl.BlockSpec((NUM_EXPERTS, 1), lambda: (0, 0)),
        ],
        out_specs=[
            pl.BlockSpec((2, N_TOKENS), lambda: (0, 0)),
            pl.BlockSpec((2, N_TOKENS), lambda: (0, 0)),
        ],
        out_shape=[
            jax.ShapeDtypeStruct((2, N_TOKENS), jnp.float32),
            jax.ShapeDtypeStruct((2, N_TOKENS), jnp.int32),
        ],
        scratch_shapes=[
            pltpu.VMEM((NBUF, CH, EMBED_DIM), jnp.float32),
            pltpu.SemaphoreType.DMA((NBUF, 2)),
        ],
    )(x, W, b.reshape(NUM_EXPERTS, 1))
    return (gates_t.T, idx_t.T)


# speedup vs baseline: 1.0658x; 1.0222x over previous
import jax
import jax.numpy as jnp
from jax import lax
from jax.experimental import pallas as pl
from jax.experimental.pallas import tpu as pltpu

EMBED_DIM = 2048
NUM_EXPERTS = 16
N_TOKENS = 16384

NBUF = 4
CH = 512
NST = N_TOKENS // CH


def _gate_body(x_hbm, w_ref, b_ref, gates_ref, idx_ref, xbuf, sems):
    w = w_ref[...]
    b2 = b_ref[...]
    HC = CH // 2

    def start(i, slot):
        pltpu.make_async_copy(
            x_hbm.at[pl.ds(i * CH, HC)], xbuf.at[slot, pl.ds(0, HC)],
            sems.at[slot, 0]).start()
        pltpu.make_async_copy(
            x_hbm.at[pl.ds(i * CH + HC, HC)], xbuf.at[slot, pl.ds(HC, HC)],
            sems.at[slot, 1]).start()

    def wait(i, slot):
        pltpu.make_async_copy(
            x_hbm.at[pl.ds(i * CH, HC)], xbuf.at[slot, pl.ds(0, HC)],
            sems.at[slot, 0]).wait()
        pltpu.make_async_copy(
            x_hbm.at[pl.ds(i * CH + HC, HC)], xbuf.at[slot, pl.ds(HC, HC)],
            sems.at[slot, 1]).wait()

    for i in range(NBUF):
        start(i, i)

    def step(i, _):
        slot = lax.rem(i, NBUF)
        wait(i, slot)
        logits = jax.lax.dot_general(
            w, xbuf[slot], (((1,), (1,)), ((), ())),
            preferred_element_type=jnp.float32) + b2

        rows = jax.lax.broadcasted_iota(jnp.int32, logits.shape, 0)
        m1 = jnp.max(logits, axis=0, keepdims=True)
        i1 = jnp.min(jnp.where(logits == m1, rows, NUM_EXPERTS),
                     axis=0, keepdims=True)
        masked = jnp.where(rows == i1, -jnp.inf, logits)
        m2 = jnp.max(masked, axis=0, keepdims=True)
        i2 = jnp.min(jnp.where(masked == m2, rows, NUM_EXPERTS),
                     axis=0, keepdims=True)
        e2 = jnp.exp(m2 - m1)
        den = 1.0 + e2
        g = jnp.concatenate([1.0 / den, e2 / den], axis=0)
        ix = jnp.concatenate([i1, i2], axis=0)
        gates_ref[:, pl.ds(i * CH, CH)] = g
        idx_ref[:, pl.ds(i * CH, CH)] = ix

        @pl.when(i + NBUF < NST)
        def _():
            start(i + NBUF, slot)

        return 0

    lax.fori_loop(0, NST, step, 0)


def kernel(x, W, b):
    gates_t, idx_t = pl.pallas_call(
        _gate_body,
        in_specs=[
            pl.BlockSpec(memory_space=pl.ANY),
            pl.BlockSpec((NUM_EXPERTS, EMBED_DIM), lambda: (0, 0)),
            pl.BlockSpec((NUM_EXPERTS, 1), lambda: (0, 0)),
        ],
        out_specs=[
            pl.BlockSpec((2, N_TOKENS), lambda: (0, 0)),
            pl.BlockSpec((2, N_TOKENS), lambda: (0, 0)),
        ],
        out_shape=[
            jax.ShapeDtypeStruct((2, N_TOKENS), jnp.float32),
            jax.ShapeDtypeStruct((2, N_TOKENS), jnp.int32),
        ],
        scratch_shapes=[
            pltpu.VMEM((NBUF, CH, EMBED_DIM), jnp.float32),
            pltpu.SemaphoreType.DMA((NBUF, 2)),
        ],
    )(x, W, b.reshape(NUM_EXPERTS, 1))
    return (gates_t.T, idx_t.T)
